# baseline (device time: 48038 ns/iter reference)
import jax
import jax.numpy as jnp
from jax import lax
from jax.experimental import pallas as pl
from jax.experimental.pallas import tpu as pltpu

NCHUNK = 8


def kernel(ids, E):
    t = ids.shape[0]
    v_local, d = E.shape
    h = t // 2
    ch = h // NCHUNK

    my_x = lax.axis_index("x")
    my_y = lax.axis_index("y")
    lo = my_x * v_local

    ids_half = lax.dynamic_slice(ids, (my_y * h,), (h,))
    local = ids_half - lo
    mask = (local >= 0) & (local < v_local)
    idx = jnp.where(mask, local, 0)
    raw = E[idx]
    maskf = mask.astype(jnp.float32)[:, None]

    def body(p_ref, m_ref, out_ref, cx_ref, sx, rx, sy, ry):
        mx = lax.axis_index("x")
        my = lax.axis_index("y")
        nbr_x = (1 - mx, my)
        nbr_y = (mx, 1 - my)

        barrier = pltpu.get_barrier_semaphore()
        for nbr in (nbr_x, nbr_y):
            pl.semaphore_signal(
                barrier, inc=1, device_id=nbr,
                device_id_type=pl.DeviceIdType.MESH,
            )
        pl.semaphore_wait(barrier, 2)

        my_off = my * h

        x_rdmas = []
        for i in range(NCHUNK):
            sl = pl.ds(i * ch, ch)
            sl_out = pl.ds(my_off + i * ch, ch)
            out_ref[sl_out, :] = p_ref[sl, :] * m_ref[sl, :]
            r = pltpu.make_async_remote_copy(
                src_ref=out_ref.at[sl_out, :],
                dst_ref=cx_ref.at[sl, :],
                send_sem=sx.at[i],
                recv_sem=rx.at[i],
                device_id=nbr_x,
                device_id_type=pl.DeviceIdType.MESH,
            )
            r.start()
            x_rdmas.append(r)

        y_rdmas = []
        for i in range(NCHUNK):
            sl = pl.ds(i * ch, ch)
            sl_out = pl.ds(my_off + i * ch, ch)
            x_rdmas[i].wait_send()
            x_rdmas[i].wait_recv()
            out_ref[sl_out, :] = out_ref[sl_out, :] + cx_ref[sl, :]
            r = pltpu.make_async_remote_copy(
                src_ref=out_ref.at[sl_out, :],
                dst_ref=out_ref.at[sl_out, :],
                send_sem=sy.at[i],
                recv_sem=ry.at[i],
                device_id=nbr_y,
                device_id_type=pl.DeviceIdType.MESH,
            )
            r.start()
            y_rdmas.append(r)

        for i in range(NCHUNK):
            y_rdmas[i].wait_recv()
            y_rdmas[i].wait_send()

    return pl.pallas_call(
        body,
        out_shape=jax.ShapeDtypeStruct((t, d), jnp.float32),
        in_specs=[
            pl.BlockSpec(memory_space=pltpu.VMEM),
            pl.BlockSpec(memory_space=pltpu.VMEM),
        ],
        out_specs=pl.BlockSpec(memory_space=pltpu.VMEM),
        scratch_shapes=[
            pltpu.VMEM((h, d), jnp.float32),
            pltpu.SemaphoreType.DMA((NCHUNK,)),
            pltpu.SemaphoreType.DMA((NCHUNK,)),
            pltpu.SemaphoreType.DMA((NCHUNK,)),
            pltpu.SemaphoreType.DMA((NCHUNK,)),
        ],
        compiler_params=pltpu.CompilerParams(collective_id=0),
    )(raw, maskf)


# device time: 41534 ns/iter; 1.1566x vs baseline; 1.1566x over previous
import jax
import jax.numpy as jnp
from jax import lax
from jax.experimental import pallas as pl
from jax.experimental.pallas import tpu as pltpu

NCHUNK = 8


def kernel(ids, E):
    t = ids.shape[0]
    v_local, d = E.shape
    h = t // 2
    ch = h // NCHUNK

    my_x = lax.axis_index("x")
    my_y = lax.axis_index("y")
    lo = my_x * v_local

    ids_half = lax.dynamic_slice(ids, (my_y * h,), (h,))
    local = ids_half - lo
    mask = (local >= 0) & (local < v_local)
    idx = jnp.where(mask, local, 0).astype(jnp.int32)
    maskf = mask.astype(jnp.float32)[:, None]

    def body(idx_ref, m_ref, e_ref, out_ref, cx_ref,
             pg0, pg1, pg2, pg3, pg4, pg5, pg6, pg7,
             gsems, sx, rx, sy, ry):
        bufs = [pg0, pg1, pg2, pg3, pg4, pg5, pg6, pg7]
        mx = lax.axis_index("x")
        my = lax.axis_index("y")
        nbr_x = (1 - mx, my)
        nbr_y = (mx, 1 - my)
        my_off = my * h

        def issue_gathers(i):
            buf = bufs[i]
            for j in range(ch):
                pltpu.make_async_copy(
                    e_ref.at[pl.ds(idx_ref[i * ch + j], 1), :],
                    buf.at[pl.ds(j, 1), :],
                    gsems.at[i],
                ).start()

        for i in range(NCHUNK):
            issue_gathers(i)

        barrier = pltpu.get_barrier_semaphore()
        for nbr in (nbr_x, nbr_y):
            pl.semaphore_signal(
                barrier, inc=1, device_id=nbr,
                device_id_type=pl.DeviceIdType.MESH,
            )
        pl.semaphore_wait(barrier, 2)

        x_rdmas = []
        for i in range(NCHUNK):
            pl.semaphore_wait(gsems.at[i], ch)
            r = pltpu.make_async_remote_copy(
                src_ref=bufs[i],
                dst_ref=cx_ref.at[pl.ds(i * ch, ch), :],
                send_sem=sx.at[i],
                recv_sem=rx.at[i],
                device_id=nbr_x,
                device_id_type=pl.DeviceIdType.MESH,
            )
            r.start()
            x_rdmas.append(r)

        y_rdmas = []
        for i in range(NCHUNK):
            sl = pl.ds(i * ch, ch)
            sl_out = pl.ds(my_off + i * ch, ch)
            x_rdmas[i].wait_recv()
            out_ref[sl_out, :] = jnp.where(
                m_ref[sl, :] > 0, bufs[i][:, :], cx_ref[sl, :]
            )
            r = pltpu.make_async_remote_copy(
                src_ref=out_ref.at[sl_out, :],
                dst_ref=out_ref.at[sl_out, :],
                send_sem=sy.at[i],
                recv_sem=ry.at[i],
                device_id=nbr_y,
                device_id_type=pl.DeviceIdType.MESH,
            )
            r.start()
            y_rdmas.append(r)

        for i in range(NCHUNK):
            x_rdmas[i].wait_send()
            y_rdmas[i].wait_recv()
            y_rdmas[i].wait_send()

    return pl.pallas_call(
        body,
        out_shape=jax.ShapeDtypeStruct((t, d), jnp.float32),
        in_specs=[
            pl.BlockSpec(memory_space=pltpu.SMEM),
            pl.BlockSpec(memory_space=pltpu.VMEM),
            pl.BlockSpec(memory_space=pltpu.MemorySpace.HBM),
        ],
        out_specs=pl.BlockSpec(memory_space=pltpu.VMEM),
        scratch_shapes=[pltpu.VMEM((h, d), jnp.float32)]
        + [pltpu.VMEM((ch, d), jnp.float32) for _ in range(NCHUNK)]
        + [
            pltpu.SemaphoreType.REGULAR((NCHUNK,)),
            pltpu.SemaphoreType.DMA((NCHUNK,)),
            pltpu.SemaphoreType.DMA((NCHUNK,)),
            pltpu.SemaphoreType.DMA((NCHUNK,)),
            pltpu.SemaphoreType.DMA((NCHUNK,)),
        ],
        compiler_params=pltpu.CompilerParams(collective_id=0),
    )(idx, maskf, E)
